# Initial kernel scaffold; baseline (speedup 1.0000x reference)
import jax
import jax.numpy as jnp
from jax.experimental import pallas as pl


def kernel(feature, edge_weight, W_fc0, b_fc0, W_conv, W_fc1, b_fc1, edge_index):
    raise NotImplementedError("write your pallas kernel here")


# speedup vs baseline: 5.0175x; 5.0175x over previous
import functools
import math

import jax
import jax.numpy as jnp
from jax import lax
from jax.experimental import pallas as pl
from jax.experimental.pallas import tpu as pltpu
from jax.experimental.pallas import tpu_sc as plsc

LAMDA = 0.5
ALPHA = 0.1

NC = 2
NS = 16
CHUNK = 128


def _spmm_body(n_chunks, n_pad, h_hbm, src_hbm, dst_hbm, w_hbm, out_hbm,
               src_v, dst_v, w_v, rows_v, acc_sh, sem):
  c = lax.axis_index("c")
  s = lax.axis_index("s")
  wid = c * NS + s

  def _zrow(e, _):
    for k in range(8):
      rows_v[e, pl.ds(k * 16, 16)] = jnp.zeros((16,), jnp.float32)
    return 0
  lax.fori_loop(0, CHUNK, _zrow, 0)
  rows_per_tile = n_pad // NS
  base = s * rows_per_tile
  for j in range(rows_per_tile // CHUNK):
    pltpu.sync_copy(rows_v, acc_sh.at[pl.ds(base + j * CHUNK, CHUNK)])
  plsc.subcore_barrier()

  n_extra = n_chunks - (n_chunks // (NC * NS)) * (NC * NS)
  my_chunks = n_chunks // (NC * NS) + jnp.where(wid < n_extra, 1, 0)

  def _edge_chunk(j, _):
    e0 = (wid + j * (NC * NS)) * CHUNK
    pltpu.sync_copy(src_hbm.at[pl.ds(e0, CHUNK)], src_v)
    pltpu.sync_copy(dst_hbm.at[pl.ds(e0, CHUNK)], dst_v)
    pltpu.sync_copy(w_hbm.at[pl.ds(e0, CHUNK)], w_v)
    pltpu.async_copy(h_hbm.at[src_v], rows_v, sem).wait()

    def _scale(g, _):
      wv = w_v[pl.ds(g * 16, 16)]
      for i in range(16):
        e = g * 16 + i
        wb = lax.gather(
            wv, jnp.full((16, 1), i, jnp.int32),
            lax.GatherDimensionNumbers(offset_dims=(),
                                       collapsed_slice_dims=(0,),
                                       start_index_map=(0,)),
            slice_sizes=(1,),
            mode=lax.GatherScatterMode.PROMISE_IN_BOUNDS)
        for k in range(8):
          sl = pl.ds(k * 16, 16)
          rows_v[e, sl] = rows_v[e, sl] * wb
      return 0
    lax.fori_loop(0, CHUNK // 16, _scale, 0)

    pltpu.sync_copy(rows_v, acc_sh.at[dst_v], add=True)
    return 0
  lax.fori_loop(0, my_chunks, _edge_chunk, 0)
  plsc.subcore_barrier()

  out_base = s * rows_per_tile
  pltpu.sync_copy(acc_sh.at[pl.ds(out_base, rows_per_tile)],
                  out_hbm.at[c, pl.ds(out_base, rows_per_tile)])


def _make_spmm(n_pad, n_edges, d):
  assert n_edges % CHUNK == 0 and n_pad % (NS * CHUNK) == 0
  n_chunks = n_edges // CHUNK
  mesh = plsc.VectorSubcoreMesh(core_axis_name="c", subcore_axis_name="s",
                                num_cores=NC, num_subcores=NS)
  return pl.kernel(
      functools.partial(_spmm_body, n_chunks, n_pad),
      out_type=jax.ShapeDtypeStruct((NC, n_pad, d), jnp.float32),
      mesh=mesh,
      scratch_types=[
          pltpu.VMEM((CHUNK,), jnp.int32),
          pltpu.VMEM((CHUNK,), jnp.int32),
          pltpu.VMEM((CHUNK,), jnp.float32),
          pltpu.VMEM((CHUNK, d), jnp.float32),
          pltpu.VMEM_SHARED((n_pad, d), jnp.float32),
          pltpu.SemaphoreType.DMA,
      ],
  )


def _fc0_body(x_ref, w_ref, b_ref, o_ref):
  h = jnp.dot(x_ref[...], w_ref[...], preferred_element_type=jnp.float32)
  o_ref[...] = jnp.maximum(h + b_ref[...], 0.0)


def _layer_body(beta, p_ref, h0_ref, w_ref, o_ref):
  hs = p_ref[0] + p_ref[1]
  support = (1.0 - ALPHA) * hs + ALPHA * h0_ref[...]
  sw = jnp.dot(support, w_ref[...], preferred_element_type=jnp.float32)
  o_ref[...] = jnp.maximum((1.0 - beta) * support + beta * sw, 0.0)


def _fc1_body(x_ref, w_ref, b_ref, o_ref):
  h = jnp.dot(x_ref[...], w_ref[...], preferred_element_type=jnp.float32)
  h = h + b_ref[...]
  m = jnp.max(h, axis=1, keepdims=True)
  hm = h - m
  o_ref[...] = hm - jnp.log(jnp.sum(jnp.exp(hm), axis=1, keepdims=True))


def _row_blocked(body, n, d_in, d_out, blk, n_in_extra_specs):
  grid = n // blk
  in_specs = [pl.BlockSpec((blk, d_in), lambda i: (i, 0))] + n_in_extra_specs
  return pl.pallas_call(
      body,
      grid=(grid,),
      in_specs=in_specs,
      out_specs=pl.BlockSpec((blk, d_out), lambda i: (i, 0)),
      out_shape=jax.ShapeDtypeStruct((n, d_out), jnp.float32),
  )


def kernel(feature, edge_weight, W_fc0, b_fc0, W_conv, W_fc1, b_fc1,
           edge_index):
  n, d_in = feature.shape
  hid = W_fc0.shape[1]
  out_d = W_fc1.shape[1]
  n_layers = W_conv.shape[0]
  e = edge_weight.shape[0]

  dst = edge_index[0].astype(jnp.int32)
  src = edge_index[1].astype(jnp.int32)
  w = edge_weight.astype(jnp.float32)

  blk = 2000
  n_pad = NS * CHUNK * pl.cdiv(n, NS * CHUNK)
  fc0 = _row_blocked(
      _fc0_body, n, d_in, hid, blk,
      [pl.BlockSpec((d_in, hid), lambda i: (0, 0)),
       pl.BlockSpec((1, hid), lambda i: (0, 0))])
  h = fc0(feature, W_fc0, b_fc0.reshape(1, hid))

  spmm = _make_spmm(n_pad, e, hid)

  for l in range(1, n_layers + 1):
    beta = math.log(LAMDA / l + 1.0)
    partial = spmm(h, src, dst, w)
    layer = pl.pallas_call(
        functools.partial(_layer_body, beta),
        grid=(n // blk,),
        in_specs=[
            pl.BlockSpec((NC, blk, hid), lambda i: (0, i, 0)),
            pl.BlockSpec((blk, hid), lambda i: (i, 0)),
            pl.BlockSpec((hid, hid), lambda i: (0, 0)),
        ],
        out_specs=pl.BlockSpec((blk, hid), lambda i: (i, 0)),
        out_shape=jax.ShapeDtypeStruct((n, hid), jnp.float32),
    )
    if l == 1:
      h0 = h
    h = layer(partial, h0, W_conv[l - 1])

  fc1 = _row_blocked(
      _fc1_body, n, hid, out_d, blk,
      [pl.BlockSpec((hid, out_d), lambda i: (0, 0)),
       pl.BlockSpec((1, out_d), lambda i: (0, 0))])
  return fc1(h, W_fc1, b_fc1.reshape(1, out_d))


# speedup vs baseline: 9.0547x; 1.8046x over previous
import functools
import math

import jax
import jax.numpy as jnp
from jax import lax
from jax.experimental import pallas as pl
from jax.experimental.pallas import tpu as pltpu
from jax.experimental.pallas import tpu_sc as plsc

LAMDA = 0.5
ALPHA = 0.1

NC = 2
NS = 16
CHUNK = 128
NBUF = 3
NJMAX = 88


def _bcast16(vec, i):
  return lax.gather(
      vec, jnp.full((16, 1), i, jnp.int32),
      lax.GatherDimensionNumbers(offset_dims=(), collapsed_slice_dims=(0,),
                                 start_index_map=(0,)),
      slice_sizes=(1,), mode=lax.GatherScatterMode.PROMISE_IN_BOUNDS)


def _spmm_body(n_chunks, n_nodes, h_hbm, src_hbm, dst_hbm, w_hbm, out_hbm,
               src0, src1, src2, dst0, dst1, dst2, w0, w1, w2,
               rows0, rows1, rows2, acc_sh, semz, semi, semg, sems):
  c = lax.axis_index("c")
  s = lax.axis_index("s")
  wid = c * NS + s
  rows = (rows0, rows1, rows2)
  srcs = (src0, src1, src2)
  dsts = (dst0, dst1, dst2)
  ws = (w0, w1, w2)

  c0 = n_chunks * wid // (NC * NS)
  count = n_chunks * (wid + 1) // (NC * NS) - c0

  def _zrow(e, _):
    for k in range(8):
      rows0[e, pl.ds(k * 16, 16)] = jnp.zeros((16,), jnp.float32)
    return 0
  lax.fori_loop(0, CHUNK, _zrow, 0)

  rpt = (n_nodes // NS) - (n_nodes // NS) % 8
  base = s * rpt
  gz = [pltpu.make_async_copy(rows0, acc_sh.at[pl.ds(base + j * CHUNK, CHUNK)],
                              semz) for j in range(rpt // CHUNK)]
  for g in gz:
    g.start()
    g.wait()

  rem15 = n_nodes - 15 * rpt - 4 * CHUNK
  rem = rpt - 4 * CHUNK

  @pl.when(s == NS - 1)
  def _ztail15():
    pltpu.sync_copy(rows0.at[pl.ds(0, rem15)],
                    acc_sh.at[pl.ds(base + 4 * CHUNK, rem15)])

  @pl.when(s != NS - 1)
  def _ztail():
    pltpu.sync_copy(rows0.at[pl.ds(0, rem)],
                    acc_sh.at[pl.ds(base + 4 * CHUNK, rem)])

  plsc.subcore_barrier()

  def _idx(k, b):
    return (pltpu.make_async_copy(src_hbm.at[pl.ds((c0 + k) * CHUNK, CHUNK)],
                                  srcs[b], semi.at[b]),
            pltpu.make_async_copy(dst_hbm.at[pl.ds((c0 + k) * CHUNK, CHUNK)],
                                  dsts[b], semi.at[b]),
            pltpu.make_async_copy(w_hbm.at[pl.ds((c0 + k) * CHUNK, CHUNK)],
                                  ws[b], semi.at[b]))

  def _idx_start(k, b):
    for g in _idx(k, b):
      g.start()

  def _idx_wait(k, b):
    for g in _idx(k, b):
      g.wait()

  def _gather(k, b):
    return pltpu.make_async_copy(h_hbm.at[srcs[b]], rows[b], semg.at[b])

  def _scatter_start(k, b):
    pltpu.async_copy(rows[b], acc_sh.at[dsts[b]], sems.at[b], add=True)

  def _scatter_wait(k, b):
    pltpu.make_async_copy(rows[b], acc_sh.at[dsts[b]], sems.at[b]).wait()

  _idx_start(0, 0)
  _idx_start(1, 1)
  _idx_wait(0, 0)
  _gather(0, 0).start()

  n_iters = ((n_chunks + NC * NS - 1) // (NC * NS) + 1 + NBUF) // NBUF

  def _outer(g, _):
    for b in range(NBUF):
      j = g * NBUF + b

      @pl.when(j < count)
      def _work():
        _gather(j, b).wait()

        def _sc(grp, _):
          wv = ws[b][pl.ds(grp * 16, 16)]
          for i in range(16):
            wb = _bcast16(wv, i)
            e = grp * 16 + i
            for k in range(8):
              sl = pl.ds(k * 16, 16)
              rows[b][e, sl] = rows[b][e, sl] * wb
          return 0
        lax.fori_loop(0, CHUNK // 16, _sc, 0)
        _scatter_start(j, b)

      @pl.when((j >= 1) & (j <= count))
      def _drain():
        _scatter_wait(j - 1, (b - 1) % NBUF)

      @pl.when(j + 2 < count)
      def _pref_idx():
        _idx_start(j + 2, (b + 2) % NBUF)

      @pl.when(j + 1 < count)
      def _pref_gather():
        _idx_wait(j + 1, (b + 1) % NBUF)
        _gather(j + 1, (b + 1) % NBUF).start()
    return 0
  lax.fori_loop(0, n_iters, _outer, 0)
  plsc.subcore_barrier()

  @pl.when(s == NS - 1)
  def _wtail15():
    pltpu.sync_copy(acc_sh.at[pl.ds(base, 5 * CHUNK)],
                    out_hbm.at[c, pl.ds(base, 5 * CHUNK)])

  @pl.when(s != NS - 1)
  def _wtail():
    pltpu.sync_copy(acc_sh.at[pl.ds(base, rpt)],
                    out_hbm.at[c, pl.ds(base, rpt)])


def _make_spmm(n_nodes, n_edges, d):
  assert n_edges % CHUNK == 0
  n_chunks = n_edges // CHUNK
  mesh = plsc.VectorSubcoreMesh(core_axis_name="c", subcore_axis_name="s",
                                num_cores=NC, num_subcores=NS)
  idx_t = [pltpu.VMEM((CHUNK,), jnp.int32) for _ in range(2 * NBUF)]
  w_t = [pltpu.VMEM((CHUNK,), jnp.float32) for _ in range(NBUF)]
  rows_t = [pltpu.VMEM((CHUNK, d), jnp.float32) for _ in range(NBUF)]
  return pl.kernel(
      functools.partial(_spmm_body, n_chunks, n_nodes),
      out_type=jax.ShapeDtypeStruct((NC, n_nodes, d), jnp.float32),
      mesh=mesh,
      scratch_types=idx_t + w_t + rows_t + [
          pltpu.VMEM_SHARED((n_nodes, d), jnp.float32),
          pltpu.SemaphoreType.DMA,
          pltpu.SemaphoreType.DMA((NBUF,)),
          pltpu.SemaphoreType.DMA((NBUF,)),
          pltpu.SemaphoreType.DMA((NBUF,)),
      ],
  )


def _fc0_body(x_ref, w_ref, b_ref, o_ref):
  h = jnp.dot(x_ref[...], w_ref[...], preferred_element_type=jnp.float32)
  o_ref[...] = jnp.maximum(h + b_ref[...], 0.0)


def _layer_body(beta, p_ref, h0_ref, w_ref, o_ref):
  hs = p_ref[0] + p_ref[1]
  support = (1.0 - ALPHA) * hs + ALPHA * h0_ref[...]
  sw = jnp.dot(support, w_ref[...], preferred_element_type=jnp.float32)
  o_ref[...] = jnp.maximum((1.0 - beta) * support + beta * sw, 0.0)


def _fc1_body(x_ref, w_ref, b_ref, o_ref):
  h = jnp.dot(x_ref[...], w_ref[...], preferred_element_type=jnp.float32)
  h = h + b_ref[...]
  m = jnp.max(h, axis=1, keepdims=True)
  hm = h - m
  o_ref[...] = hm - jnp.log(jnp.sum(jnp.exp(hm), axis=1, keepdims=True))


def _row_blocked(body, n, d_in, d_out, blk, n_in_extra_specs):
  grid = n // blk
  in_specs = [pl.BlockSpec((blk, d_in), lambda i: (i, 0))] + n_in_extra_specs
  return pl.pallas_call(
      body,
      grid=(grid,),
      in_specs=in_specs,
      out_specs=pl.BlockSpec((blk, d_out), lambda i: (i, 0)),
      out_shape=jax.ShapeDtypeStruct((n, d_out), jnp.float32),
  )


def kernel(feature, edge_weight, W_fc0, b_fc0, W_conv, W_fc1, b_fc1,
           edge_index):
  n, d_in = feature.shape
  hid = W_fc0.shape[1]
  out_d = W_fc1.shape[1]
  n_layers = W_conv.shape[0]
  e = edge_weight.shape[0]

  dst = edge_index[0].astype(jnp.int32)
  src = edge_index[1].astype(jnp.int32)
  w = edge_weight.astype(jnp.float32)

  blk = 2000
  fc0 = _row_blocked(
      _fc0_body, n, d_in, hid, blk,
      [pl.BlockSpec((d_in, hid), lambda i: (0, 0)),
       pl.BlockSpec((1, hid), lambda i: (0, 0))])
  h = fc0(feature, W_fc0, b_fc0.reshape(1, hid))

  spmm = _make_spmm(n, e, hid)

  for l in range(1, n_layers + 1):
    beta = math.log(LAMDA / l + 1.0)
    partial = spmm(h, src, dst, w)
    layer = pl.pallas_call(
        functools.partial(_layer_body, beta),
        grid=(n // blk,),
        in_specs=[
            pl.BlockSpec((NC, blk, hid), lambda i: (0, i, 0)),
            pl.BlockSpec((blk, hid), lambda i: (i, 0)),
            pl.BlockSpec((hid, hid), lambda i: (0, 0)),
        ],
        out_specs=pl.BlockSpec((blk, hid), lambda i: (i, 0)),
        out_shape=jax.ShapeDtypeStruct((n, hid), jnp.float32),
    )
    if l == 1:
      h0 = h
    h = layer(partial, h0, W_conv[l - 1])

  fc1 = _row_blocked(
      _fc1_body, n, hid, out_d, blk,
      [pl.BlockSpec((hid, out_d), lambda i: (0, 0)),
       pl.BlockSpec((1, out_d), lambda i: (0, 0))])
  return fc1(h, W_fc1, b_fc1.reshape(1, out_d))
